# baseline (speedup 1.0000x reference)
import jax
import jax.numpy as jnp
from jax import lax
from jax.experimental import pallas as pl
from jax.experimental.pallas import tpu as pltpu
from jax.experimental.pallas import tpu_sc as plsc

N = 32768
C = 128
G = 64

_ROWS = 8192
_NT = 16
_E = N // _NT
_L = 16


def _tc_body(x_ref, ids_ref, s1_ref):
    i = pl.program_id(0)
    x = x_ref[...]
    m = jnp.max(x, axis=1, keepdims=True)
    e = jnp.exp(x - m)
    s = jnp.dot(e, jnp.ones((C, 1), jnp.float32),
                preferred_element_type=jnp.float32)
    logz = jnp.log(s) + m
    ids = ids_ref[...]
    gi = lax.broadcasted_iota(jnp.int32, (G, _ROWS), 0)
    oh_t = jnp.where(gi == ids[None, :], 1.0, 0.0)
    r = jnp.dot(oh_t, logz, preferred_element_type=jnp.float32)

    @pl.when(i == 0)
    def _():
        s1_ref[...] = jnp.zeros((G,), jnp.float32)

    s1_ref[...] += r[:, 0]


def _logz_group_sums(y_pred, ids):
    grid = N // _ROWS
    return pl.pallas_call(
        _tc_body,
        grid=(grid,),
        in_specs=[
            pl.BlockSpec((_ROWS, C), lambda i: (i, 0)),
            pl.BlockSpec((_ROWS,), lambda i: (i,)),
        ],
        out_specs=pl.BlockSpec((G,), lambda i: (0,)),
        out_shape=jax.ShapeDtypeStruct((G,), jnp.float32),
    )(y_pred, ids)


def _gather_body(yp_hbm, yt_hbm, ids_hbm, s2c_hbm,
                 yt_v, idx_v, tl_v, ids_v, sums_v, cnts_v, comb_v, big_v,
                 shared, sem):
    s = lax.axis_index("s")
    base = s * _E
    pltpu.sync_copy(yt_hbm.at[pl.ds(base, _E)], yt_v)
    pltpu.sync_copy(ids_hbm.at[pl.ds(base, _E)], ids_v)

    lane = lax.iota(jnp.int32, _L)

    def ibody(i, carry):
        row = base + i * _L + lane
        idx_v[pl.ds(i * _L, _L)] = row * C + yt_v[pl.ds(i * _L, _L)]
        return carry

    lax.fori_loop(0, _E // _L, ibody, 0)

    cp = pltpu.async_copy(yp_hbm.at[idx_v], tl_v, sem)

    zeros = jnp.zeros((_L,), jnp.float32)
    for r in range(_L):
        for j in range(G // _L):
            sums_v[pl.ds(r * G + j * _L, _L)] = zeros
            cnts_v[pl.ds(r * G + j * _L, _L)] = zeros

    cp.wait()

    lane_off = lane * G
    ones = jnp.ones((_L,), jnp.float32)

    def body(i, carry):
        t = tl_v[pl.ds(i * _L, _L)]
        g = ids_v[pl.ds(i * _L, _L)]
        plsc.addupdate_scatter(sums_v, [lane_off + g], t)
        plsc.addupdate_scatter(cnts_v, [lane_off + g], ones)
        return carry

    lax.fori_loop(0, _E // _L, body, 0)

    for j in range(G // _L):
        acc_s = sums_v[pl.ds(j * _L, _L)]
        acc_c = cnts_v[pl.ds(j * _L, _L)]
        for r in range(1, _L):
            acc_s = acc_s + sums_v[pl.ds(r * G + j * _L, _L)]
            acc_c = acc_c + cnts_v[pl.ds(r * G + j * _L, _L)]
        comb_v[pl.ds(j * _L, _L)] = acc_s
        comb_v[pl.ds(G + j * _L, _L)] = acc_c

    pltpu.sync_copy(comb_v, shared.at[pl.ds(s * 2 * G, 2 * G)])
    plsc.subcore_barrier()

    @pl.when(s == 0)
    def _():
        pltpu.sync_copy(shared, big_v)
        for j in range(2 * G // _L):
            acc = big_v[pl.ds(j * _L, _L)]
            for t in range(1, _NT):
                acc = acc + big_v[pl.ds(t * 2 * G + j * _L, _L)]
            comb_v[pl.ds(j * _L, _L)] = acc
        pltpu.sync_copy(comb_v, s2c_hbm)


def _gather_s2_cnt(yp_flat, y_true, ids):
    mesh = plsc.VectorSubcoreMesh(
        core_axis_name="c", subcore_axis_name="s", num_cores=1)
    kfn = pl.kernel(
        _gather_body,
        out_type=jax.ShapeDtypeStruct((2 * G,), jnp.float32),
        mesh=mesh,
        compiler_params=pltpu.CompilerParams(needs_layout_passes=False),
        scratch_types=[
            pltpu.VMEM((_E,), jnp.int32),
            pltpu.VMEM((_E,), jnp.int32),
            pltpu.VMEM((_E,), jnp.float32),
            pltpu.VMEM((_E,), jnp.int32),
            pltpu.VMEM((_L * G,), jnp.float32),
            pltpu.VMEM((_L * G,), jnp.float32),
            pltpu.VMEM((2 * G,), jnp.float32),
            pltpu.VMEM((_NT * 2 * G,), jnp.float32),
            pltpu.VMEM_SHARED((_NT * 2 * G,), jnp.float32),
            pltpu.SemaphoreType.DMA,
        ],
    )
    return kfn(yp_flat, y_true, ids)


def _final_body(s1_ref, s2c_ref, out_ref):
    s1 = s1_ref[...]
    s2 = s2c_ref[0:G]
    cv = s2c_ref[G:2 * G]
    gl = jnp.where(cv > 0.0, (s1 - s2) / jnp.maximum(cv, 1.0), 0.0)
    out_ref[...] = jnp.full((1,), jnp.sum(gl), jnp.float32)


def _final(s1, s2c):
    return pl.pallas_call(
        _final_body,
        out_shape=jax.ShapeDtypeStruct((1,), jnp.float32),
    )(s1, s2c)


def kernel(y_pred, y_true, id_mask):
    ids = id_mask.reshape(N).astype(jnp.int32)
    s2c = _gather_s2_cnt(y_pred.reshape(N * C),
                         y_true.reshape(N).astype(jnp.int32), ids)
    s1 = _logz_group_sums(y_pred, ids)
    out1 = _final(s1, s2c)
    return out1[0]


# speedup vs baseline: 1.0259x; 1.0259x over previous
import jax
import jax.numpy as jnp
from jax import lax
from jax.experimental import pallas as pl
from jax.experimental.pallas import tpu as pltpu
from jax.experimental.pallas import tpu_sc as plsc

N = 32768
C = 128
G = 64

_ROWS = 8192
_NT = 16
_E = N // _NT
_L = 16


def _tc_body(x_ref, ids_ref, s1_ref, cnt_ref):
    i = pl.program_id(0)
    x = x_ref[...]
    m = jnp.max(x, axis=1, keepdims=True)
    e = jnp.exp(x - m)
    s = jnp.sum(e, axis=1, keepdims=True)
    logz = jnp.log(s) + m
    ids = ids_ref[...]
    gi = lax.broadcasted_iota(jnp.int32, (G, _ROWS), 0)
    oh_t = jnp.where(gi == ids[None, :], 1.0, 0.0)
    b = jnp.concatenate([logz, jnp.ones_like(logz)], axis=1)
    r = jnp.dot(oh_t, b, preferred_element_type=jnp.float32)

    @pl.when(i == 0)
    def _():
        s1_ref[...] = jnp.zeros((G,), jnp.float32)
        cnt_ref[...] = jnp.zeros((G,), jnp.float32)

    s1_ref[...] += r[:, 0]
    cnt_ref[...] += r[:, 1]


def _logz_group_sums(y_pred, ids):
    grid = N // _ROWS
    return pl.pallas_call(
        _tc_body,
        grid=(grid,),
        in_specs=[
            pl.BlockSpec((_ROWS, C), lambda i: (i, 0)),
            pl.BlockSpec((_ROWS,), lambda i: (i,)),
        ],
        out_specs=[
            pl.BlockSpec((G,), lambda i: (0,)),
            pl.BlockSpec((G,), lambda i: (0,)),
        ],
        out_shape=[
            jax.ShapeDtypeStruct((G,), jnp.float32),
            jax.ShapeDtypeStruct((G,), jnp.float32),
        ],
    )(y_pred, ids)


def _gather_body(yp_hbm, yt_hbm, ids_hbm, s2_hbm,
                 yt_v, idx_v, tl_v, ids_v, sums_v, comb_v, big_v,
                 shared, sem):
    s = lax.axis_index("s")
    base = s * _E
    pltpu.sync_copy(yt_hbm.at[pl.ds(base, _E)], yt_v)
    pltpu.sync_copy(ids_hbm.at[pl.ds(base, _E)], ids_v)

    lane = lax.iota(jnp.int32, _L)

    def ibody(i, carry):
        row = base + i * _L + lane
        idx_v[pl.ds(i * _L, _L)] = row * C + yt_v[pl.ds(i * _L, _L)]
        return carry

    lax.fori_loop(0, _E // _L, ibody, 0)

    cp = pltpu.async_copy(yp_hbm.at[idx_v], tl_v, sem)

    zeros = jnp.zeros((_L,), jnp.float32)
    for r in range(_L):
        for j in range(G // _L):
            sums_v[pl.ds(r * G + j * _L, _L)] = zeros

    cp.wait()

    lane_off = lane * G

    def body(i, carry):
        t = tl_v[pl.ds(i * _L, _L)]
        g = ids_v[pl.ds(i * _L, _L)]
        plsc.addupdate_scatter(sums_v, [lane_off + g], t)
        return carry

    lax.fori_loop(0, _E // _L, body, 0)

    for j in range(G // _L):
        acc = sums_v[pl.ds(j * _L, _L)]
        for r in range(1, _L):
            acc = acc + sums_v[pl.ds(r * G + j * _L, _L)]
        comb_v[pl.ds(j * _L, _L)] = acc

    pltpu.sync_copy(comb_v, shared.at[pl.ds(s * G, G)])
    plsc.subcore_barrier()

    @pl.when(s == 0)
    def _():
        pltpu.sync_copy(shared, big_v)
        for j in range(G // _L):
            s2 = big_v[pl.ds(j * _L, _L)]
            for t in range(1, _NT):
                s2 = s2 + big_v[pl.ds(t * G + j * _L, _L)]
            comb_v[pl.ds(j * _L, _L)] = s2
        pltpu.sync_copy(comb_v, s2_hbm)


def _gather_s2(yp_flat, y_true, ids):
    mesh = plsc.VectorSubcoreMesh(
        core_axis_name="c", subcore_axis_name="s", num_cores=1)
    kfn = pl.kernel(
        _gather_body,
        out_type=jax.ShapeDtypeStruct((G,), jnp.float32),
        mesh=mesh,
        compiler_params=pltpu.CompilerParams(needs_layout_passes=False),
        scratch_types=[
            pltpu.VMEM((_E,), jnp.int32),
            pltpu.VMEM((_E,), jnp.int32),
            pltpu.VMEM((_E,), jnp.float32),
            pltpu.VMEM((_E,), jnp.int32),
            pltpu.VMEM((_L * G,), jnp.float32),
            pltpu.VMEM((G,), jnp.float32),
            pltpu.VMEM((_NT * G,), jnp.float32),
            pltpu.VMEM_SHARED((_NT * G,), jnp.float32),
            pltpu.SemaphoreType.DMA,
        ],
    )
    return kfn(yp_flat, y_true, ids)


def _final_body(s1_ref, cnt_ref, s2_ref, out_ref):
    s1 = s1_ref[...]
    cv = cnt_ref[...]
    s2 = s2_ref[...]
    gl = jnp.where(cv > 0.0, (s1 - s2) / jnp.maximum(cv, 1.0), 0.0)
    out_ref[...] = jnp.full((1,), jnp.sum(gl), jnp.float32)


def _final(s1, cnt, s2):
    return pl.pallas_call(
        _final_body,
        out_shape=jax.ShapeDtypeStruct((1,), jnp.float32),
    )(s1, cnt, s2)


def kernel(y_pred, y_true, id_mask):
    ids = id_mask.reshape(N).astype(jnp.int32)
    s2 = _gather_s2(y_pred.reshape(N * C),
                    y_true.reshape(N).astype(jnp.int32), ids)
    s1, cnt = _logz_group_sums(y_pred, ids)
    out1 = _final(s1, cnt, s2)
    return out1[0]
